# baseline (speedup 1.0000x reference)
import functools

import jax
import jax.numpy as jnp
from jax import lax
from jax.experimental import pallas as pl
from jax.experimental.pallas import tpu as pltpu
from jax.experimental.pallas import tpu_sc as plsc

BATCH = 4
SEQ = 4096
D = 1024
LANES = 16
NUM_CORES = 2
NUM_SUBCORES = 16
NUM_WORKERS = NUM_CORES * NUM_SUBCORES
ROWS_PER_WORKER = SEQ // NUM_WORKERS
CHUNK_ROWS = 16
CHUNKS = ROWS_PER_WORKER // CHUNK_ROWS
CHUNK_ELEMS = CHUNK_ROWS * D
VECS_PER_CHUNK = CHUNK_ELEMS // LANES
VECS_PER_ROW = D // LANES
NSTAGES = CHUNKS * BATCH
NBUF = 5


def _make_sc_kernel():
    mesh = plsc.VectorSubcoreMesh(core_axis_name="c", subcore_axis_name="s")

    @functools.partial(
        pl.kernel,
        mesh=mesh,
        out_type=jax.ShapeDtypeStruct((BATCH, SEQ, D), jnp.float32),
        compiler_params=pltpu.CompilerParams(use_tc_tiling_on_sc=True),
        scratch_types=[
            pltpu.VMEM((2, CHUNK_ROWS, D), jnp.float32),
            pltpu.VMEM((NBUF, CHUNK_ROWS, D), jnp.float32),
        ]
        + [pltpu.SemaphoreType.DMA] * NBUF
        + [pltpu.SemaphoreType.DMA] * NBUF
        + [pltpu.SemaphoreType.DMA] * 2,
    )
    def sc_add(x_hbm, t_hbm, out_hbm, tbuf, xbuf, *sems):
        isems = sems[0:NBUF]
        osems = sems[NBUF:2 * NBUF]
        tsems = sems[2 * NBUF:]
        wid = lax.axis_index("s") * NUM_CORES + lax.axis_index("c")
        s_base = wid * ROWS_PER_WORKER

        def row0(stage):
            ci = stage // BATCH
            return s_base + ci * CHUNK_ROWS

        def start_in(stage):
            b = stage % BATCH
            return pltpu.async_copy(
                x_hbm.at[b, pl.ds(row0(stage), CHUNK_ROWS)],
                xbuf.at[stage % NBUF], isems[stage % NBUF])

        HALF_ROWS = CHUNK_ROWS // 2

        def start_out_half(stage, h):
            b = stage % BATCH
            return pltpu.async_copy(
                xbuf.at[stage % NBUF, pl.ds(h * HALF_ROWS, HALF_ROWS)],
                out_hbm.at[b, pl.ds(row0(stage) + h * HALF_ROWS, HALF_ROWS)],
                osems[stage % NBUF])

        def start_t(ci):
            return pltpu.async_copy(
                t_hbm.at[pl.ds(s_base + ci * CHUNK_ROWS, CHUNK_ROWS)],
                tbuf.at[ci % 2], tsems[ci % 2])

        t_copies = [start_t(0), start_t(1)]
        in_copies = {s: start_in(s) for s in range(min(NBUF - 1, NSTAGES))}
        out_copies = {}

        for stage in range(NSTAGES):
            ci, b = divmod(stage, BATCH)
            if b == 0:
                t_copies[ci % 2].wait()
            in_copies.pop(stage).wait()

            tb = tbuf.at[ci % 2]
            xb = xbuf.at[stage % NBUF]

            stage_outs = []
            for h in range(2):
                base = h * (VECS_PER_CHUNK // 2)

                @plsc.parallel_loop(base, base + VECS_PER_CHUNK // 2, unroll=8)
                def _(i):
                    r = i // VECS_PER_ROW
                    c = (i % VECS_PER_ROW) * LANES
                    sl = pl.ds(c, LANES)
                    xb[r, sl] = xb[r, sl] + tb[r, sl]

                stage_outs.append(start_out_half(stage, h))
            out_copies[stage] = stage_outs
            nxt = stage + NBUF - 1
            if nxt < NSTAGES:
                if stage - 1 in out_copies:
                    for oc in out_copies.pop(stage - 1):
                        oc.wait()
                in_copies[nxt] = start_in(nxt)
            if b == BATCH - 1 and ci + 2 < CHUNKS:
                t_copies[ci % 2] = start_t(ci + 2)

        for s in sorted(out_copies):
            for oc in out_copies[s]:
                oc.wait()

    return sc_add


_SC_ADD = _make_sc_kernel()


@jax.jit
def kernel(x, pos_table):
    return _SC_ADD(x, pos_table)


# speedup vs baseline: 1.0403x; 1.0403x over previous
import functools

import jax
import jax.numpy as jnp
from jax import lax
from jax.experimental import pallas as pl
from jax.experimental.pallas import tpu as pltpu
from jax.experimental.pallas import tpu_sc as plsc

BATCH = 4
SEQ = 4096
D = 1024
LANES = 16
NUM_CORES = 2
NUM_SUBCORES = 16
NUM_WORKERS = NUM_CORES * NUM_SUBCORES
ROWS_PER_WORKER = SEQ // NUM_WORKERS
CHUNK_ROWS = 16
CHUNKS = ROWS_PER_WORKER // CHUNK_ROWS
CHUNK_ELEMS = CHUNK_ROWS * D
VECS_PER_CHUNK = CHUNK_ELEMS // LANES
VECS_PER_ROW = D // LANES
NSTAGES = CHUNKS * BATCH
NBUF = 5


def _make_sc_kernel():
    mesh = plsc.VectorSubcoreMesh(core_axis_name="c", subcore_axis_name="s")

    @functools.partial(
        pl.kernel,
        mesh=mesh,
        out_type=jax.ShapeDtypeStruct((BATCH, SEQ, D), jnp.float32),
        compiler_params=pltpu.CompilerParams(use_tc_tiling_on_sc=True),
        scratch_types=[
            pltpu.VMEM((2, CHUNK_ROWS, D), jnp.float32),
            pltpu.VMEM((NBUF, CHUNK_ROWS, D), jnp.float32),
        ]
        + [pltpu.SemaphoreType.DMA] * NBUF
        + [pltpu.SemaphoreType.DMA] * NBUF
        + [pltpu.SemaphoreType.DMA] * 2,
    )
    def sc_add(x_hbm, t_hbm, out_hbm, tbuf, xbuf, *sems):
        isems = sems[0:NBUF]
        osems = sems[NBUF:2 * NBUF]
        tsems = sems[2 * NBUF:]
        wid = lax.axis_index("s") * NUM_CORES + lax.axis_index("c")
        s_base = wid * ROWS_PER_WORKER

        def row0(stage):
            ci = stage // BATCH
            return s_base + ci * CHUNK_ROWS

        def start_in(stage):
            b = stage % BATCH
            return pltpu.async_copy(
                x_hbm.at[b, pl.ds(row0(stage), CHUNK_ROWS)],
                xbuf.at[stage % NBUF], isems[stage % NBUF])

        def start_out(stage):
            b = stage % BATCH
            return pltpu.async_copy(
                xbuf.at[stage % NBUF],
                out_hbm.at[b, pl.ds(row0(stage), CHUNK_ROWS)],
                osems[stage % NBUF])

        def start_t(ci):
            return pltpu.async_copy(
                t_hbm.at[pl.ds(s_base + ci * CHUNK_ROWS, CHUNK_ROWS)],
                tbuf.at[ci % 2], tsems[ci % 2])

        t_copies = [start_t(0), start_t(1)]
        in_copies = {s: start_in(s) for s in range(min(NBUF - 1, NSTAGES))}
        out_copies = {}

        for stage in range(NSTAGES):
            ci, b = divmod(stage, BATCH)
            if b == 0:
                t_copies[ci % 2].wait()
            in_copies.pop(stage).wait()

            tb = tbuf.at[ci % 2]
            xb = xbuf.at[stage % NBUF]

            @plsc.parallel_loop(0, VECS_PER_CHUNK, unroll=8)
            def _(i):
                r = i // VECS_PER_ROW
                c = (i % VECS_PER_ROW) * LANES
                sl = pl.ds(c, LANES)
                xb[r, sl] = xb[r, sl] + tb[r, sl]

            out_copies[stage] = [start_out(stage)]
            nxt = stage + NBUF - 1
            if nxt < NSTAGES:
                if stage - 1 in out_copies:
                    for oc in out_copies.pop(stage - 1):
                        oc.wait()
                in_copies[nxt] = start_in(nxt)
            if b == BATCH - 1 and ci + 2 < CHUNKS:
                t_copies[ci % 2] = start_t(ci + 2)

        for s in sorted(out_copies):
            for oc in out_copies[s]:
                oc.wait()

    return sc_add


_SC_ADD = _make_sc_kernel()


@jax.jit
def kernel(x, pos_table):
    return _SC_ADD(x, pos_table)


# speedup vs baseline: 1.0528x; 1.0120x over previous
import functools

import jax
import jax.numpy as jnp
from jax import lax
from jax.experimental import pallas as pl
from jax.experimental.pallas import tpu as pltpu
from jax.experimental.pallas import tpu_sc as plsc

BATCH = 4
SEQ = 4096
D = 1024
LANES = 16
NUM_CORES = 2
NUM_SUBCORES = 16
NUM_WORKERS = NUM_CORES * NUM_SUBCORES
ROWS_PER_WORKER = SEQ // NUM_WORKERS
CHUNK_ROWS = 16
CHUNKS = ROWS_PER_WORKER // CHUNK_ROWS
CHUNK_ELEMS = CHUNK_ROWS * D
VECS_PER_CHUNK = CHUNK_ELEMS // LANES
VECS_PER_ROW = D // LANES
NSTAGES = CHUNKS * BATCH
NBUF = 5


def _make_sc_kernel():
    mesh = plsc.VectorSubcoreMesh(core_axis_name="c", subcore_axis_name="s")

    @functools.partial(
        pl.kernel,
        mesh=mesh,
        out_type=jax.ShapeDtypeStruct((BATCH, SEQ, D), jnp.float32),
        compiler_params=pltpu.CompilerParams(use_tc_tiling_on_sc=True),
        scratch_types=[
            pltpu.VMEM((2, CHUNK_ROWS, D), jnp.float32),
            pltpu.VMEM((NBUF, CHUNK_ROWS, D), jnp.float32),
        ]
        + [pltpu.SemaphoreType.DMA] * NBUF
        + [pltpu.SemaphoreType.DMA] * NBUF
        + [pltpu.SemaphoreType.DMA] * 2,
    )
    def sc_add(x_hbm, t_hbm, out_hbm, tbuf, xbuf, *sems):
        isems = sems[0:NBUF]
        osems = sems[NBUF:2 * NBUF]
        tsems = sems[2 * NBUF:]
        wid = lax.axis_index("s") * NUM_CORES + lax.axis_index("c")
        s_base = wid * ROWS_PER_WORKER

        def row0(stage):
            ci = stage // BATCH
            return s_base + ci * CHUNK_ROWS

        def start_in(stage):
            b = stage % BATCH
            return pltpu.async_copy(
                x_hbm.at[b, pl.ds(row0(stage), CHUNK_ROWS)],
                xbuf.at[stage % NBUF], isems[stage % NBUF])

        def start_out(stage):
            b = stage % BATCH
            return pltpu.async_copy(
                xbuf.at[stage % NBUF],
                out_hbm.at[b, pl.ds(row0(stage), CHUNK_ROWS)],
                osems[stage % NBUF])

        def start_t(ci):
            return pltpu.async_copy(
                t_hbm.at[pl.ds(s_base + ci * CHUNK_ROWS, CHUNK_ROWS)],
                tbuf.at[ci % 2], tsems[ci % 2])

        t_copies = [start_t(0), start_t(1)]
        in_copies = {s: start_in(s) for s in range(min(NBUF - 1, NSTAGES))}
        out_copies = {}

        for stage in range(NSTAGES):
            ci, b = divmod(stage, BATCH)
            if b == 0:
                t_copies[ci % 2].wait()
            in_copies.pop(stage).wait()

            tb = tbuf.at[ci % 2]
            xb = xbuf.at[stage % NBUF]

            @plsc.parallel_loop(0, VECS_PER_CHUNK, unroll=8)
            def _(i):
                r = i // VECS_PER_ROW
                c = (i % VECS_PER_ROW) * LANES
                sl = pl.ds(c, LANES)
                plsc.addupdate(xb.at[r, sl], tb[r, sl])

            out_copies[stage] = [start_out(stage)]
            nxt = stage + NBUF - 1
            if nxt < NSTAGES:
                if stage - 1 in out_copies:
                    for oc in out_copies.pop(stage - 1):
                        oc.wait()
                in_copies[nxt] = start_in(nxt)
            if b == BATCH - 1 and ci + 2 < CHUNKS:
                t_copies[ci % 2] = start_t(ci + 2)

        for s in sorted(out_copies):
            for oc in out_copies[s]:
                oc.wait()

    return sc_add


_SC_ADD = _make_sc_kernel()


@jax.jit
def kernel(x, pos_table):
    return _SC_ADD(x, pos_table)
